# baseline (speedup 1.0000x reference)
import functools

import jax
import jax.numpy as jnp
from jax.experimental import pallas as pl
from jax.experimental.pallas import tpu as pltpu

_ALPHA = 16.0
_NT = (((1,), (1,)), ((), ()))


def _moe_lora_kernel(x_ref, w_ref, gate_ref, a_ref, bmat_ref, out_ref,
                     lhs_ref, wcat_ref, stage_ref, sem_ref, *, E, r, H):
    @pl.when(pl.program_id(0) == 0)
    def _build_wcat():
        O = wcat_ref.shape[0]
        n_chunks = 4
        ch = O // n_chunks
        copies = [
            pltpu.make_async_copy(w_ref.at[pl.ds(k * ch, ch), :],
                                  stage_ref.at[k % 2], sem_ref.at[k % 2])
            for k in range(n_chunks)
        ]
        copies[0].start()
        copies[1].start()
        wcat_ref[:, H:] = bmat_ref[...]
        for k in range(n_chunks):
            copies[k].wait()
            wcat_ref[pl.ds(k * ch, ch), :H] = stage_ref[k % 2].astype(jnp.bfloat16)
            if k + 2 < n_chunks:
                copies[k + 2].start()

    xt = x_ref[...]
    xb = xt.astype(jnp.bfloat16)
    logits = jax.lax.dot_general(xt, gate_ref[...], _NT,
                                 preferred_element_type=jnp.float32)
    h = jax.lax.dot_general(xb, a_ref[...], _NT,
                            preferred_element_type=jnp.float32)

    T = logits.shape[0]
    eid = jax.lax.broadcasted_iota(jnp.int32, (T, E), 1)
    m1 = jnp.max(logits, axis=1, keepdims=True)
    i1 = jnp.min(jnp.where(logits == m1, eid, E), axis=1, keepdims=True)
    masked = jnp.where(eid == i1, -jnp.inf, logits)
    m2 = jnp.max(masked, axis=1, keepdims=True)
    i2 = jnp.min(jnp.where(masked == m2, eid, E), axis=1, keepdims=True)
    w2 = 1.0 / (1.0 + jnp.exp(m1 - m2))
    w1 = 1.0 - w2

    ke = jax.lax.broadcasted_iota(jnp.int32, (T, E * r), 1) // r
    c = jnp.where(ke == i1, w1, 0.0) + jnp.where(ke == i2, w2, 0.0)
    lhs_ref[:, :H] = xb
    lhs_ref[:, H:] = (h * (c * _ALPHA)).astype(jnp.bfloat16)
    out_ref[...] = jax.lax.dot_general(lhs_ref[...], wcat_ref[...], _NT,
                                       preferred_element_type=jnp.float32)


def kernel(x, W, gate_W, As, Bs):
    B, S, H = x.shape
    O = W.shape[0]
    E, r, _ = As.shape
    N = B * S
    xf = x.reshape(N, H)
    A_flat = As.reshape(E * r, H).astype(jnp.bfloat16)
    Bmat = (jnp.transpose(Bs, (1, 0, 2)).reshape(O, E * r)
            .astype(jnp.bfloat16))

    T = 1024
    K = H + E * r
    grid = (N // T,)
    out = pl.pallas_call(
        functools.partial(_moe_lora_kernel, E=E, r=r, H=H),
        grid=grid,
        in_specs=[
            pl.BlockSpec((T, H), lambda i: (i, 0)),
            pl.BlockSpec(memory_space=pl.ANY),
            pl.BlockSpec((E, H), lambda i: (0, 0)),
            pl.BlockSpec((E * r, H), lambda i: (0, 0)),
            pl.BlockSpec((O, E * r), lambda i: (0, 0)),
        ],
        out_specs=pl.BlockSpec((T, O), lambda i: (i, 0)),
        out_shape=jax.ShapeDtypeStruct((N, O), jnp.float32),
        scratch_shapes=[
            pltpu.VMEM((T, K), jnp.bfloat16),
            pltpu.VMEM((O, K), jnp.bfloat16),
            pltpu.VMEM((2, O // 4, H), jnp.float32),
            pltpu.SemaphoreType.DMA((2,)),
        ],
        compiler_params=pltpu.CompilerParams(
            vmem_limit_bytes=100 * 1024 * 1024,
        ),
    )(xf, W, gate_W, A_flat, Bmat)
    return out.reshape(B, S, O)


# speedup vs baseline: 1.0379x; 1.0379x over previous
import functools

import jax
import jax.numpy as jnp
from jax.experimental import pallas as pl
from jax.experimental.pallas import tpu as pltpu

_ALPHA = 16.0
_NT = (((1,), (1,)), ((), ()))


def _moe_lora_kernel(x_ref, w_ref, gate_ref, a_ref, bmat_ref, out_ref,
                     lhs_ref, wcat_ref, stage_ref, sem_ref, *, E, r, H):
    @pl.when(pl.program_id(0) == 0)
    def _build_wcat():
        O = wcat_ref.shape[0]
        n_chunks = 4
        ch = O // n_chunks
        copies = [
            pltpu.make_async_copy(w_ref.at[pl.ds(k * ch, ch), :],
                                  stage_ref.at[k % 2], sem_ref.at[k % 2])
            for k in range(n_chunks)
        ]
        copies[0].start()
        copies[1].start()
        wcat_ref[:, H:] = bmat_ref[...]
        for k in range(n_chunks):
            copies[k].wait()
            wcat_ref[pl.ds(k * ch, ch), :H] = stage_ref[k % 2].astype(jnp.bfloat16)
            if k + 2 < n_chunks:
                copies[k + 2].start()

    xt = x_ref[...]
    xb = xt.astype(jnp.bfloat16)
    logits = jax.lax.dot_general(xt, gate_ref[...], _NT,
                                 preferred_element_type=jnp.float32)
    h = jax.lax.dot_general(xb, a_ref[...], _NT,
                            preferred_element_type=jnp.float32)

    T = logits.shape[0]
    eid = jax.lax.broadcasted_iota(jnp.int32, (T, E), 1)
    m1 = jnp.max(logits, axis=1, keepdims=True)
    i1 = jnp.min(jnp.where(logits == m1, eid, E), axis=1, keepdims=True)
    masked = jnp.where(eid == i1, -jnp.inf, logits)
    m2 = jnp.max(masked, axis=1, keepdims=True)
    i2 = jnp.min(jnp.where(masked == m2, eid, E), axis=1, keepdims=True)
    w2 = 1.0 / (1.0 + jnp.exp(m1 - m2))
    w1 = 1.0 - w2

    ke = jax.lax.broadcasted_iota(jnp.int32, (T, E * r), 1) // r
    c = jnp.where(ke == i1, w1, 0.0) + jnp.where(ke == i2, w2, 0.0)
    lhs_ref[:, :H] = xb
    lhs_ref[:, H:] = (h * (c * _ALPHA)).astype(jnp.bfloat16)
    out_ref[...] = jax.lax.dot_general(lhs_ref[...], wcat_ref[...], _NT,
                                       preferred_element_type=jnp.float32)


def kernel(x, W, gate_W, As, Bs):
    B, S, H = x.shape
    O = W.shape[0]
    E, r, _ = As.shape
    N = B * S
    xf = x.reshape(N, H)
    A_flat = As.reshape(E * r, H).astype(jnp.bfloat16)
    Bmat = (jnp.transpose(Bs, (1, 0, 2)).reshape(O, E * r)
            .astype(jnp.bfloat16))

    T = 512
    K = H + E * r
    grid = (N // T,)
    out = pl.pallas_call(
        functools.partial(_moe_lora_kernel, E=E, r=r, H=H),
        grid=grid,
        in_specs=[
            pl.BlockSpec((T, H), lambda i: (i, 0)),
            pl.BlockSpec(memory_space=pl.ANY),
            pl.BlockSpec((E, H), lambda i: (0, 0)),
            pl.BlockSpec((E * r, H), lambda i: (0, 0)),
            pl.BlockSpec((O, E * r), lambda i: (0, 0)),
        ],
        out_specs=pl.BlockSpec((T, O), lambda i: (i, 0)),
        out_shape=jax.ShapeDtypeStruct((N, O), jnp.float32),
        scratch_shapes=[
            pltpu.VMEM((T, K), jnp.bfloat16),
            pltpu.VMEM((O, K), jnp.bfloat16),
            pltpu.VMEM((2, O // 4, H), jnp.float32),
            pltpu.SemaphoreType.DMA((2,)),
        ],
        compiler_params=pltpu.CompilerParams(
            vmem_limit_bytes=100 * 1024 * 1024,
        ),
    )(xf, W, gate_W, A_flat, Bmat)
    return out.reshape(B, S, O)


# speedup vs baseline: 1.0461x; 1.0079x over previous
import functools

import jax
import jax.numpy as jnp
from jax.experimental import pallas as pl
from jax.experimental.pallas import tpu as pltpu

_ALPHA = 16.0
_NT = (((1,), (1,)), ((), ()))


def _moe_lora_kernel(x_ref, w_ref, gate_ref, a_ref, bmat_ref, out_ref,
                     lhs_ref, wcat_ref, *, E, r, H):
    @pl.when(pl.program_id(0) == 0)
    def _build_wcat():
        wcat_ref[:, :H] = w_ref[...].astype(jnp.bfloat16)
        wcat_ref[:, H:] = bmat_ref[...]

    xt = x_ref[...]
    xb = xt.astype(jnp.bfloat16)
    logits = jax.lax.dot_general(xt, gate_ref[...], _NT,
                                 preferred_element_type=jnp.float32)
    h = jax.lax.dot_general(xb, a_ref[...], _NT,
                            preferred_element_type=jnp.float32)

    T = logits.shape[0]
    eid = jax.lax.broadcasted_iota(jnp.int32, (T, E), 1)
    m1 = jnp.max(logits, axis=1, keepdims=True)
    i1 = jnp.min(jnp.where(logits == m1, eid, E), axis=1, keepdims=True)
    masked = jnp.where(eid == i1, -jnp.inf, logits)
    m2 = jnp.max(masked, axis=1, keepdims=True)
    i2 = jnp.min(jnp.where(masked == m2, eid, E), axis=1, keepdims=True)
    w2 = 1.0 / (1.0 + jnp.exp(m1 - m2))
    w1 = 1.0 - w2

    ke = jax.lax.broadcasted_iota(jnp.int32, (T, E * r), 1) // r
    c = jnp.where(ke == i1, w1, 0.0) + jnp.where(ke == i2, w2, 0.0)
    lhs_ref[:, :H] = xb
    lhs_ref[:, H:] = (h * (c * _ALPHA)).astype(jnp.bfloat16)
    out_ref[...] = jax.lax.dot_general(lhs_ref[...], wcat_ref[...], _NT,
                                       preferred_element_type=jnp.float32)


def kernel(x, W, gate_W, As, Bs):
    B, S, H = x.shape
    O = W.shape[0]
    E, r, _ = As.shape
    N = B * S
    xf = x.reshape(N, H)
    A_flat = As.reshape(E * r, H).astype(jnp.bfloat16)
    Bmat = (jnp.transpose(Bs, (1, 0, 2)).reshape(O, E * r)
            .astype(jnp.bfloat16))

    T = 512
    K = H + E * r
    grid = (N // T,)
    out = pl.pallas_call(
        functools.partial(_moe_lora_kernel, E=E, r=r, H=H),
        grid=grid,
        in_specs=[
            pl.BlockSpec((T, H), lambda i: (i, 0)),
            pl.BlockSpec((O, H), lambda i: (0, 0)),
            pl.BlockSpec((E, H), lambda i: (0, 0)),
            pl.BlockSpec((E * r, H), lambda i: (0, 0)),
            pl.BlockSpec((O, E * r), lambda i: (0, 0)),
        ],
        out_specs=pl.BlockSpec((T, O), lambda i: (i, 0)),
        out_shape=jax.ShapeDtypeStruct((N, O), jnp.float32),
        scratch_shapes=[
            pltpu.VMEM((T, K), jnp.bfloat16),
            pltpu.VMEM((O, K), jnp.bfloat16),
        ],
        compiler_params=pltpu.CompilerParams(
            vmem_limit_bytes=100 * 1024 * 1024,
        ),
    )(xf, W, gate_W, A_flat, Bmat)
    return out.reshape(B, S, O)


# speedup vs baseline: 1.0516x; 1.0053x over previous
import functools

import jax
import jax.numpy as jnp
from jax.experimental import pallas as pl
from jax.experimental.pallas import tpu as pltpu

_ALPHA = 16.0
_NT = (((1,), (1,)), ((), ()))


def _moe_lora_kernel(x_ref, w_ref, gate_ref, a_ref, bmat_ref, out_ref,
                     lhs_ref, wcat_ref, *, E, r, H):
    @pl.when(pl.program_id(0) == 0)
    def _build_wcat():
        wcat_ref[:, :H] = w_ref[...].astype(jnp.bfloat16)
        wcat_ref[:, H:] = bmat_ref[...]

    xt = x_ref[...]
    xb = xt.astype(jnp.bfloat16)
    lhs_ref[:, :H] = xb
    logits = jax.lax.dot_general(xt, gate_ref[...], _NT,
                                 preferred_element_type=jnp.float32)
    h = jax.lax.dot_general(xb, a_ref[...], _NT,
                            preferred_element_type=jnp.float32)

    T = logits.shape[0]
    eid = jax.lax.broadcasted_iota(jnp.int32, (T, E), 1)
    m1 = jnp.max(logits, axis=1, keepdims=True)
    i1 = jnp.min(jnp.where(logits == m1, eid, E), axis=1, keepdims=True)
    masked = jnp.where(eid == i1, -jnp.inf, logits)
    m2 = jnp.max(masked, axis=1, keepdims=True)
    i2 = jnp.min(jnp.where(masked == m2, eid, E), axis=1, keepdims=True)
    w2 = 1.0 / (1.0 + jnp.exp(m1 - m2))
    w1 = 1.0 - w2

    ke = jax.lax.broadcasted_iota(jnp.int32, (T, E * r), 1) // r
    c = jnp.where(ke == i1, w1, 0.0) + jnp.where(ke == i2, w2, 0.0)
    lhs_ref[:, H:] = (h * (c * _ALPHA)).astype(jnp.bfloat16)
    out_ref[...] = jax.lax.dot_general(lhs_ref[...], wcat_ref[...], _NT,
                                       preferred_element_type=jnp.float32)


def kernel(x, W, gate_W, As, Bs):
    B, S, H = x.shape
    O = W.shape[0]
    E, r, _ = As.shape
    N = B * S
    xf = x.reshape(N, H)
    A_flat = As.reshape(E * r, H).astype(jnp.bfloat16)
    Bmat = (jnp.transpose(Bs, (1, 0, 2)).reshape(O, E * r)
            .astype(jnp.bfloat16))

    T = 512
    K = H + E * r
    grid = (N // T,)
    out = pl.pallas_call(
        functools.partial(_moe_lora_kernel, E=E, r=r, H=H),
        grid=grid,
        in_specs=[
            pl.BlockSpec((T, H), lambda i: (i, 0)),
            pl.BlockSpec((O, H), lambda i: (0, 0)),
            pl.BlockSpec((E, H), lambda i: (0, 0)),
            pl.BlockSpec((E * r, H), lambda i: (0, 0)),
            pl.BlockSpec((O, E * r), lambda i: (0, 0)),
        ],
        out_specs=pl.BlockSpec((T, O), lambda i: (i, 0)),
        out_shape=jax.ShapeDtypeStruct((N, O), jnp.float32),
        scratch_shapes=[
            pltpu.VMEM((T, K), jnp.bfloat16),
            pltpu.VMEM((O, K), jnp.bfloat16),
        ],
        compiler_params=pltpu.CompilerParams(
            vmem_limit_bytes=100 * 1024 * 1024,
        ),
    )(xf, W, gate_W, A_flat, Bmat)
    return out.reshape(B, S, O)
